# baseline (speedup 1.0000x reference)
import functools

import jax
import jax.numpy as jnp
from jax import lax
from jax.experimental import pallas as pl
from jax.experimental.pallas import tpu as pltpu
from jax.experimental.pallas import tpu_sc as plsc

_STRIP = 1024


@functools.lru_cache(maxsize=None)
def _build(N, S, D):
    info = plsc.get_sparse_core_info()
    nc = info.num_cores
    nw = nc * info.num_subcores
    B = N * S
    n_strips = B // _STRIP
    strips_per_w = n_strips // nw
    assert n_strips % nw == 0 and N % (4 * 128) == 0 and D == 32

    mesh = plsc.VectorSubcoreMesh(core_axis_name="c", subcore_axis_name="s")

    @functools.partial(
        pl.kernel,
        mesh=mesh,
        compiler_params=pltpu.CompilerParams(
            use_tc_tiling_on_sc=False, needs_layout_passes=False),
        out_type=jax.ShapeDtypeStruct((B * D,), jnp.float32),
        scratch_types=[
            pltpu.VMEM((_STRIP,), jnp.int32),
            pltpu.VMEM((_STRIP,), jnp.int32),
            pltpu.VMEM((_STRIP, D), jnp.float32),
            pltpu.VMEM((_STRIP, D), jnp.float32),
            pltpu.VMEM((_STRIP * D,), jnp.float32),
            pltpu.SemaphoreType.DMA,
            pltpu.SemaphoreType.DMA,
        ],
    )
    def gather_kernel(idx_hbm, table_hbm, out_hbm,
                      idx0, idx1, rows0, rows1, o_v, sem0, sem1):
        wid = lax.axis_index("s") * nc + lax.axis_index("c")
        u0 = wid * strips_per_w
        lanes = lax.iota(jnp.int32, 16)
        idx_bufs = (idx0, idx1)
        row_bufs = (rows0, rows1)
        sems = (sem0, sem1)

        def start(j):
            idx_v, rows_v, sem = idx_bufs[j % 2], row_bufs[j % 2], sems[j % 2]
            pltpu.sync_copy(idx_hbm.at[pl.ds((u0 + j) * _STRIP, _STRIP)], idx_v)
            return pltpu.async_copy(table_hbm.at[idx_v], rows_v, sem)

        def transpose_strip(rows_v):
            def body(g, _):
                i = g >> 6
                t = (g >> 3) & 7
                sub = g & 7
                col = jnp.broadcast_to(i * 8 + sub, (16,))
                for l0 in range(0, 128, 16):
                    row = t * 128 + l0 + lanes
                    v = plsc.load_gather(rows_v, [row, col])
                    o_v[pl.ds(g * 128 + l0, 16)] = v
                return 0

            lax.fori_loop(0, 256, body, 0)

        def writeout(j):
            u = u0 + j
            s_id = u // 4
            q = u % 4
            for i in range(4):
                off = (128 * s_id + 32 * i + 8 * q) * 1024
                pltpu.sync_copy(o_v.at[pl.ds(i * 8192, 8192)],
                                out_hbm.at[pl.ds(off, 8192)])

        copies = [start(0), None]
        for j in range(strips_per_w):
            if j + 1 < strips_per_w:
                copies[(j + 1) % 2] = start(j + 1)
            copies[j % 2].wait()
            transpose_strip(row_bufs[j % 2])
            writeout(j)

    return gather_kernel


def kernel(token_ids, weight):
    n, s = token_ids.shape
    d = weight.shape[1]
    idx = token_ids.T.reshape(n * s).astype(jnp.int32)
    out = _build(n, s, d)(idx, weight)
    o4 = out.reshape(s, d // 8, n // 128, 8, 128)
    return o4.transpose(2, 4, 0, 1, 3).reshape(n, s, d)


# speedup vs baseline: 1.0039x; 1.0039x over previous
import functools

import jax
import jax.numpy as jnp
from jax import lax
from jax.experimental import pallas as pl
from jax.experimental.pallas import tpu as pltpu
from jax.experimental.pallas import tpu_sc as plsc

_STRIP = 1024


@functools.lru_cache(maxsize=None)
def _build(N, S, D):
    info = plsc.get_sparse_core_info()
    nc = info.num_cores
    nw = nc * info.num_subcores
    B = N * S
    n_strips = B // _STRIP
    strips_per_w = n_strips // nw
    assert n_strips % nw == 0 and N % (4 * 128) == 0 and D == 32

    mesh = plsc.VectorSubcoreMesh(core_axis_name="c", subcore_axis_name="s")

    @functools.partial(
        pl.kernel,
        mesh=mesh,
        compiler_params=pltpu.CompilerParams(
            use_tc_tiling_on_sc=False, needs_layout_passes=False),
        out_type=jax.ShapeDtypeStruct((B * D,), jnp.float32),
        scratch_types=[
            pltpu.VMEM((_STRIP,), jnp.int32),
            pltpu.VMEM((_STRIP,), jnp.int32),
            pltpu.VMEM((_STRIP, D), jnp.float32),
            pltpu.VMEM((_STRIP, D), jnp.float32),
            pltpu.VMEM((_STRIP * D,), jnp.float32),
            pltpu.SemaphoreType.DMA,
            pltpu.SemaphoreType.DMA,
        ],
    )
    def gather_kernel(idx_hbm, table_hbm, out_hbm,
                      idx0, idx1, rows0, rows1, o_v, sem0, sem1):
        wid = lax.axis_index("s") * nc + lax.axis_index("c")
        u0 = wid * strips_per_w
        lanes = lax.iota(jnp.int32, 16)
        cols = [jnp.broadcast_to(jnp.int32(c), (16,)) for c in range(D)]
        idx_bufs = (idx0, idx1)
        row_bufs = (rows0, rows1)
        sems = (sem0, sem1)

        def start(j, b):
            idx_v, rows_v, sem = idx_bufs[b], row_bufs[b], sems[b]
            off = pl.multiple_of((u0 + j) * _STRIP, _STRIP)
            pltpu.sync_copy(idx_hbm.at[pl.ds(off, _STRIP)], idx_v)
            pltpu.async_copy(table_hbm.at[idx_v], rows_v, sem)

        def wait(b):
            pltpu.make_async_copy(
                table_hbm.at[idx_bufs[b]], row_bufs[b], sems[b]).wait()

        def transpose_strip(rows_v):
            def body(g, _):
                row = g * 16 + lanes
                m16 = g * 16
                for c in range(D):
                    v = plsc.load_gather(rows_v, [row, cols[c]])
                    o_v[pl.ds((c // 8) * 8192 + (c % 8) * 128 + m16 // 128 * 1024
                              + m16 % 128, 16)] = v
                return 0

            lax.fori_loop(0, 64, body, 0)

        def writeout(j):
            u = u0 + j
            s_id = u // 4
            q = u % 4
            for i in range(4):
                off = pl.multiple_of(
                    (128 * s_id + 32 * i + 8 * q) * 1024, 8192)
                pltpu.sync_copy(o_v.at[pl.ds(i * 8192, 8192)],
                                out_hbm.at[pl.ds(off, 8192)])

        start(0, 0)
        start(1, 1)

        def ring_body(k, _):
            for b in range(2):
                j = k * 2 + b
                wait(b)
                transpose_strip(row_bufs[b])
                writeout(j)

                @pl.when(j + 2 < strips_per_w)
                def _():
                    start(j + 2, b)

            return 0

        lax.fori_loop(0, strips_per_w // 2, ring_body, 0)
        if strips_per_w % 2:
            j_last = strips_per_w - 1
            wait(0)
            transpose_strip(row_bufs[0])
            writeout(j_last)

    return gather_kernel


def kernel(token_ids, weight):
    n, s = token_ids.shape
    d = weight.shape[1]
    idx = token_ids.T.reshape(n * s).astype(jnp.int32)
    out = _build(n, s, d)(idx, weight)
    o4 = out.reshape(s, d // 8, n // 128, 8, 128)
    return o4.transpose(2, 4, 0, 1, 3).reshape(n, s, d)


# speedup vs baseline: 1.5463x; 1.5403x over previous
import functools

import jax
import jax.numpy as jnp
from jax import lax
from jax.experimental import pallas as pl
from jax.experimental.pallas import tpu as pltpu
from jax.experimental.pallas import tpu_sc as plsc

_STRIP = 1024


@functools.lru_cache(maxsize=None)
def _build(N, S, D):
    info = plsc.get_sparse_core_info()
    nc = info.num_cores
    nw = nc * info.num_subcores
    B = N * S
    n_strips = B // _STRIP
    strips_per_w = n_strips // nw
    assert n_strips % nw == 0 and N % (4 * 128) == 0 and D == 32

    mesh = plsc.VectorSubcoreMesh(core_axis_name="c", subcore_axis_name="s")

    @functools.partial(
        pl.kernel,
        mesh=mesh,
        compiler_params=pltpu.CompilerParams(
            use_tc_tiling_on_sc=False, needs_layout_passes=False),
        out_type=jax.ShapeDtypeStruct((B * D,), jnp.float32),
        scratch_types=[
            pltpu.VMEM((_STRIP,), jnp.int32),
            pltpu.VMEM((_STRIP,), jnp.int32),
            pltpu.VMEM((_STRIP, D), jnp.float32),
            pltpu.VMEM((_STRIP, D), jnp.float32),
            pltpu.VMEM((_STRIP * D,), jnp.float32),
            pltpu.SemaphoreType.DMA,
            pltpu.SemaphoreType.DMA,
        ],
    )
    def gather_kernel(idx_hbm, table_hbm, out_hbm,
                      idx0, idx1, rows0, rows1, o_v, sem0, sem1):
        wid = lax.axis_index("s") * nc + lax.axis_index("c")
        u0 = wid * strips_per_w
        lanes = lax.iota(jnp.int32, 16)
        colv = [(lanes + k) & 15 for k in range(16)]
        pf = [(cv >> 3) * 8192 + (cv & 7) * 128 for cv in colv]
        idx_bufs = (idx0, idx1)
        row_bufs = (rows0, rows1)
        sems = (sem0, sem1)

        def start(j, b):
            idx_v, rows_v, sem = idx_bufs[b], row_bufs[b], sems[b]
            off = pl.multiple_of((u0 + j) * _STRIP, _STRIP)
            pltpu.sync_copy(idx_hbm.at[pl.ds(off, _STRIP)], idx_v)
            pltpu.async_copy(table_hbm.at[idx_v], rows_v, sem)

        def wait(b):
            pltpu.make_async_copy(
                table_hbm.at[idx_bufs[b]], row_bufs[b], sems[b]).wait()

        def transpose_strip(rows_v):
            def body(g, _):
                m0 = g * 16
                tl0 = (m0 >> 7) * 1024 + (m0 & 127)
                tlvec = jnp.broadcast_to(tl0, (16,)) + lanes
                rowv = jnp.broadcast_to(m0, (16,)) + lanes
                for c0 in (0, 16):
                    for k in range(16):
                        col = colv[k] + c0 if c0 else colv[k]
                        v = plsc.load_gather(rows_v, [rowv, col])
                        pv = pf[k] + tlvec
                        if c0:
                            pv = pv + 16384
                        plsc.store_scatter(o_v, [pv], v)
                return 0

            lax.fori_loop(0, 64, body, 0)

        def writeout(j):
            u = u0 + j
            s_id = u // 4
            q = u % 4
            for i in range(4):
                off = pl.multiple_of(
                    (128 * s_id + 32 * i + 8 * q) * 1024, 8192)
                pltpu.sync_copy(o_v.at[pl.ds(i * 8192, 8192)],
                                out_hbm.at[pl.ds(off, 8192)])

        start(0, 0)
        start(1, 1)

        def ring_body(k, _):
            for b in range(2):
                j = k * 2 + b
                wait(b)
                transpose_strip(row_bufs[b])
                writeout(j)

                @pl.when(j + 2 < strips_per_w)
                def _():
                    start(j + 2, b)

            return 0

        lax.fori_loop(0, strips_per_w // 2, ring_body, 0)
        if strips_per_w % 2:
            j_last = strips_per_w - 1
            wait(0)
            transpose_strip(row_bufs[0])
            writeout(j_last)

    return gather_kernel


def kernel(token_ids, weight):
    n, s = token_ids.shape
    d = weight.shape[1]
    idx = token_ids.T.reshape(n * s).astype(jnp.int32)
    out = _build(n, s, d)(idx, weight)
    o4 = out.reshape(s, d // 8, n // 128, 8, 128)
    return o4.transpose(2, 4, 0, 1, 3).reshape(n, s, d)
